# baseline (speedup 1.0000x reference)
import jax
import jax.numpy as jnp
from jax.experimental import pallas as pl
from jax.experimental.pallas import tpu as pltpu

NUM_CLASSES = 1000
_BB = 16
_K = 6


def _body(x_ref, o_hbm, buf, sem):
    i = pl.program_id(0)
    n = pl.num_programs(0)
    slot = jax.lax.rem(i, _K)

    @pl.when(i >= _K)
    def _wait_prev():
        pltpu.make_async_copy(
            buf.at[slot], o_hbm.at[pl.ds((i - _K) * _BB, _BB)], sem.at[slot]
        ).wait()

    xi = x_ref[pl.ds(i * _BB, _BB), :]
    iota = jax.lax.broadcasted_iota(
        jnp.int32, (_BB, x_ref.shape[1], NUM_CLASSES), 2
    )
    buf[slot] = (iota == xi[:, :, None]).astype(jnp.float32)

    for k in range(_K):
        @pl.when(slot == k)
        def _start():
            pltpu.make_async_copy(
                buf.at[k], o_hbm.at[pl.ds(i * _BB, _BB)], sem.at[k]
            ).start(priority=k % 2)

    @pl.when(i == n - 1)
    def _drain():
        for k in range(_K):
            s = jax.lax.rem(i + 1 + k, _K)
            pltpu.make_async_copy(
                buf.at[s], o_hbm.at[pl.ds(0, _BB)], sem.at[s]
            ).wait()


def kernel(x):
    B, F, _ = x.shape
    xs = jnp.squeeze(x, -1)
    return pl.pallas_call(
        _body,
        grid=(B // _BB,),
        in_specs=[pl.BlockSpec((B, F), lambda i: (0, 0))],
        out_specs=pl.BlockSpec(memory_space=pl.ANY),
        out_shape=jax.ShapeDtypeStruct((B, F, NUM_CLASSES), jnp.float32),
        scratch_shapes=[
            pltpu.VMEM((_K, _BB, F, NUM_CLASSES), jnp.float32),
            pltpu.SemaphoreType.DMA((_K,)),
        ],
    )(xs)
